# Initial kernel scaffold; baseline (speedup 1.0000x reference)
import jax
import jax.numpy as jnp
from jax.experimental import pallas as pl


def kernel(x, edge_index, edge_weight, index, W1, W2):
    raise NotImplementedError("write your pallas kernel here")


# speedup vs baseline: 5.6908x; 5.6908x over previous
import functools

import jax
import jax.numpy as jnp
from jax import lax
from jax.experimental import pallas as pl
from jax.experimental.pallas import tpu as pltpu
from jax.experimental.pallas import tpu_sc as plsc

N_NODES = 10000
N_EDGES = 320000
D_IN = 128
D_HID = 16
D_OUT = 64
N_IDX = 1024
N_PAD = 10240

NC = 2
NS = 16
CH = 80

E_PER_CORE = N_EDGES // NC
E_PER_TILE = E_PER_CORE // NS
N_CHUNKS = E_PER_TILE // CH
ROWS_PER_TILE = N_PAD // NS
IDX_PER_TILE = N_IDX // (NC * NS)

_mesh = functools.partial(
    plsc.VectorSubcoreMesh,
    core_axis_name="c",
    subcore_axis_name="s",
    num_cores=NC,
    num_subcores=NS,
)


def _agg_body(src_hbm, row_hbm, col_hbm, w_hbm, out0_hbm, out1_hbm,
              colv, rowv, wv, rows_v, zbuf, acc, sem):
    c = lax.axis_index("c")
    s = lax.axis_index("s")

    def zero_row(i, carry):
        zbuf[i, :] = jnp.zeros((16,), jnp.float32)
        return carry
    lax.fori_loop(0, ROWS_PER_TILE, zero_row, 0)
    sl = pl.ds(s * ROWS_PER_TILE, ROWS_PER_TILE)
    pltpu.sync_copy(zbuf, acc.at[sl])
    plsc.subcore_barrier()

    base0 = c * E_PER_CORE + s * E_PER_TILE

    def chunk(j, carry):
        b = base0 + j * CH
        pltpu.sync_copy(col_hbm.at[pl.ds(b, CH)], colv)
        pltpu.sync_copy(row_hbm.at[pl.ds(b, CH)], rowv)
        pltpu.sync_copy(w_hbm.at[pl.ds(b, CH)], wv)
        pltpu.async_copy(src_hbm.at[colv], rows_v, sem).wait()

        def scale(g, inner):
            w16 = wv[pl.ds(g * 16, 16)]
            e0 = g * 16
            for k in range(16):
                rows_v[e0 + k, :] = rows_v[e0 + k, :] * w16[k]
            return inner
        lax.fori_loop(0, CH // 16, scale, 0)

        pltpu.sync_copy(rows_v, acc.at[rowv], add=True)
        return carry

    lax.fori_loop(0, N_CHUNKS, chunk, 0)
    plsc.subcore_barrier()

    @pl.when(c == 0)
    def _():
        pltpu.sync_copy(acc.at[sl], out0_hbm.at[sl])

    @pl.when(c == 1)
    def _():
        pltpu.sync_copy(acc.at[sl], out1_hbm.at[sl])


_agg = pl.kernel(
    _agg_body,
    out_type=(
        jax.ShapeDtypeStruct((N_PAD, D_HID), jnp.float32),
        jax.ShapeDtypeStruct((N_PAD, D_HID), jnp.float32),
    ),
    mesh=_mesh(),
    scratch_types=[
        pltpu.VMEM((CH,), jnp.int32),
        pltpu.VMEM((CH,), jnp.int32),
        pltpu.VMEM((CH,), jnp.float32),
        pltpu.VMEM((CH, D_HID), jnp.float32),
        pltpu.VMEM((ROWS_PER_TILE, D_HID), jnp.float32),
        pltpu.VMEM_SHARED((N_PAD, D_HID), jnp.float32),
        pltpu.SemaphoreType.DMA,
    ],
    compiler_params=pltpu.CompilerParams(use_tc_tiling_on_sc=False),
    name="edge_agg_sc",
)


def _fin_body(q0_hbm, q1_hbm, idx_hbm, out_hbm, idxv, a_v, b_v, sem):
    wid = lax.axis_index("s") * NC + lax.axis_index("c")
    base = wid * IDX_PER_TILE
    pltpu.sync_copy(idx_hbm.at[pl.ds(base, IDX_PER_TILE)], idxv)
    pltpu.async_copy(q0_hbm.at[idxv], a_v, sem).wait()
    pltpu.async_copy(q1_hbm.at[idxv], b_v, sem).wait()

    def add_row(i, carry):
        a_v[i, :] = a_v[i, :] + b_v[i, :]
        return carry
    lax.fori_loop(0, IDX_PER_TILE, add_row, 0)
    pltpu.sync_copy(a_v, out_hbm.at[pl.ds(base, IDX_PER_TILE)])


_fin = pl.kernel(
    _fin_body,
    out_type=jax.ShapeDtypeStruct((N_IDX, D_HID), jnp.float32),
    mesh=_mesh(),
    scratch_types=[
        pltpu.VMEM((IDX_PER_TILE,), jnp.int32),
        pltpu.VMEM((IDX_PER_TILE, D_HID), jnp.float32),
        pltpu.VMEM((IDX_PER_TILE, D_HID), jnp.float32),
        pltpu.SemaphoreType.DMA,
    ],
    compiler_params=pltpu.CompilerParams(use_tc_tiling_on_sc=False),
    name="final_gather_sc",
)


def _mm1_body(x_ref, w_ref, o_ref):
    o_ref[...] = jnp.dot(x_ref[...], w_ref[...],
                         preferred_element_type=jnp.float32)


def _mm1(x, W1):
    grid = 5
    rows = N_NODES // grid
    return pl.pallas_call(
        _mm1_body,
        grid=(grid,),
        in_specs=[
            pl.BlockSpec((rows, D_IN), lambda i: (i, 0)),
            pl.BlockSpec((D_IN, D_HID), lambda i: (0, 0)),
        ],
        out_specs=pl.BlockSpec((rows, D_HID), lambda i: (i, 0)),
        out_shape=jax.ShapeDtypeStruct((N_NODES, D_HID), jnp.float32),
    )(x, W1)


def _comb_body(a_ref, b_ref, o_ref):
    o_ref[...] = jnp.maximum(a_ref[...] + b_ref[...], 0.0)


def _comb(p0, p1):
    grid = 5
    rows = N_PAD // grid
    return pl.pallas_call(
        _comb_body,
        grid=(grid,),
        in_specs=[
            pl.BlockSpec((rows, D_HID), lambda i: (i, 0)),
            pl.BlockSpec((rows, D_HID), lambda i: (i, 0)),
        ],
        out_specs=pl.BlockSpec((rows, D_HID), lambda i: (i, 0)),
        out_shape=jax.ShapeDtypeStruct((N_PAD, D_HID), jnp.float32),
    )(p0, p1)


def _mm2_body(h_ref, w_ref, o_ref):
    o_ref[...] = jnp.dot(h_ref[...], w_ref[...],
                         preferred_element_type=jnp.float32)


def _mm2(h16, W2):
    return pl.pallas_call(
        _mm2_body,
        in_specs=[
            pl.BlockSpec((N_IDX, D_HID), lambda: (0, 0)),
            pl.BlockSpec((D_HID, D_OUT), lambda: (0, 0)),
        ],
        out_specs=pl.BlockSpec((N_IDX, D_OUT), lambda: (0, 0)),
        out_shape=jax.ShapeDtypeStruct((N_IDX, D_OUT), jnp.float32),
    )(h16, W2)


def kernel(x, edge_index, edge_weight, index, W1, W2):
    row = edge_index[:, 0]
    col = edge_index[:, 1]
    hw1 = _mm1(x, W1)
    p0, p1 = _agg(hw1, row, col, edge_weight)
    h = _comb(p0, p1)
    q0, q1 = _agg(h, row, col, edge_weight)
    out16 = _fin(q0, q1, index)
    return _mm2(out16, W2)


# speedup vs baseline: 21.9665x; 3.8600x over previous
import functools

import jax
import jax.numpy as jnp
from jax import lax
from jax.experimental import pallas as pl
from jax.experimental.pallas import tpu as pltpu
from jax.experimental.pallas import tpu_sc as plsc

N_NODES = 10000
N_EDGES = 320000
D_IN = 128
D_HID = 16
D_OUT = 64
N_IDX = 1024
N_PAD = 10240

NC = 2
NS = 16
CH = 80

E_PER_CORE = N_EDGES // NC
E_PER_TILE = E_PER_CORE // NS
N_CHUNKS = E_PER_TILE // CH
ROWS_PER_TILE = N_PAD // NS
IDX_PER_TILE = N_IDX // (NC * NS)

_mesh = functools.partial(
    plsc.VectorSubcoreMesh,
    core_axis_name="c",
    subcore_axis_name="s",
    num_cores=NC,
    num_subcores=NS,
)


NBUF = 5
N_GROUPS = N_CHUNKS // NBUF


def _agg_body(src_hbm, row_hbm, col_hbm, w_hbm, out0_hbm, out1_hbm,
              colv, roww, wv, bufs, zbuf, acc, *sems):
    gsems = sems[:NBUF]
    ssems = sems[NBUF:]
    c = lax.axis_index("c")
    s = lax.axis_index("s")

    def zero_row(i, carry):
        zbuf[i, :] = jnp.zeros((16,), jnp.float32)
        return carry
    lax.fori_loop(0, ROWS_PER_TILE, zero_row, 0)
    sl = pl.ds(s * ROWS_PER_TILE, ROWS_PER_TILE)
    pltpu.sync_copy(zbuf, acc.at[sl])

    pltpu.sync_copy(col_hbm.at[c, s], colv)
    pltpu.sync_copy(row_hbm.at[c, s], roww)
    pltpu.sync_copy(w_hbm.at[c, s], wv)
    plsc.subcore_barrier()

    def group(g, carry):
        j0 = g * NBUF
        gd = [
            pltpu.async_copy(src_hbm.at[colv.at[j0 + r]], bufs.at[r], gsems[r])
            for r in range(NBUF)
        ]
        sd = []
        for r in range(NBUF):
            gd[r].wait()

            def scale(m, inner, r=r):
                w16 = wv[j0 + r, pl.ds(m * 16, 16)]
                e0 = m * 16
                for k in range(16):
                    bufs[r, e0 + k, :] = bufs[r, e0 + k, :] * w16[k]
                return inner
            lax.fori_loop(0, CH // 16, scale, 0)
            sd.append(pltpu.async_copy(bufs.at[r], acc.at[roww.at[j0 + r]],
                                       ssems[r], add=True))
        for d in sd:
            d.wait()
        return carry

    lax.fori_loop(0, N_GROUPS, group, 0)
    plsc.subcore_barrier()

    @pl.when(c == 0)
    def _():
        pltpu.sync_copy(acc.at[sl], out0_hbm.at[sl])

    @pl.when(c == 1)
    def _():
        pltpu.sync_copy(acc.at[sl], out1_hbm.at[sl])


_agg = pl.kernel(
    _agg_body,
    out_type=(
        jax.ShapeDtypeStruct((N_PAD, D_HID), jnp.float32),
        jax.ShapeDtypeStruct((N_PAD, D_HID), jnp.float32),
    ),
    mesh=_mesh(),
    scratch_types=[
        pltpu.VMEM((N_CHUNKS, CH), jnp.int32),
        pltpu.VMEM((N_CHUNKS, CH), jnp.int32),
        pltpu.VMEM((N_CHUNKS, CH), jnp.float32),
        pltpu.VMEM((NBUF, CH, D_HID), jnp.float32),
        pltpu.VMEM((ROWS_PER_TILE, D_HID), jnp.float32),
        pltpu.VMEM_SHARED((N_PAD, D_HID), jnp.float32),
    ] + [pltpu.SemaphoreType.DMA] * (2 * NBUF),
    compiler_params=pltpu.CompilerParams(use_tc_tiling_on_sc=False),
    name="edge_agg_sc",
)


def _fin_body(q0_hbm, q1_hbm, idx_hbm, out_hbm, idxv, a_v, b_v, sem):
    wid = lax.axis_index("s") * NC + lax.axis_index("c")
    base = wid * IDX_PER_TILE
    pltpu.sync_copy(idx_hbm.at[pl.ds(base, IDX_PER_TILE)], idxv)
    pltpu.async_copy(q0_hbm.at[idxv], a_v, sem).wait()
    pltpu.async_copy(q1_hbm.at[idxv], b_v, sem).wait()

    def add_row(i, carry):
        a_v[i, :] = a_v[i, :] + b_v[i, :]
        return carry
    lax.fori_loop(0, IDX_PER_TILE, add_row, 0)
    pltpu.sync_copy(a_v, out_hbm.at[pl.ds(base, IDX_PER_TILE)])


_fin = pl.kernel(
    _fin_body,
    out_type=jax.ShapeDtypeStruct((N_IDX, D_HID), jnp.float32),
    mesh=_mesh(),
    scratch_types=[
        pltpu.VMEM((IDX_PER_TILE,), jnp.int32),
        pltpu.VMEM((IDX_PER_TILE, D_HID), jnp.float32),
        pltpu.VMEM((IDX_PER_TILE, D_HID), jnp.float32),
        pltpu.SemaphoreType.DMA,
    ],
    compiler_params=pltpu.CompilerParams(use_tc_tiling_on_sc=False),
    name="final_gather_sc",
)


def _mm1_body(x_ref, w_ref, o_ref):
    o_ref[...] = jnp.dot(x_ref[...], w_ref[...],
                         preferred_element_type=jnp.float32)


def _mm1(x, W1):
    grid = 5
    rows = N_NODES // grid
    return pl.pallas_call(
        _mm1_body,
        grid=(grid,),
        in_specs=[
            pl.BlockSpec((rows, D_IN), lambda i: (i, 0)),
            pl.BlockSpec((D_IN, D_HID), lambda i: (0, 0)),
        ],
        out_specs=pl.BlockSpec((rows, D_HID), lambda i: (i, 0)),
        out_shape=jax.ShapeDtypeStruct((N_NODES, D_HID), jnp.float32),
    )(x, W1)


def _comb_body(a_ref, b_ref, o_ref):
    o_ref[...] = jnp.maximum(a_ref[...] + b_ref[...], 0.0)


def _comb(p0, p1):
    grid = 5
    rows = N_PAD // grid
    return pl.pallas_call(
        _comb_body,
        grid=(grid,),
        in_specs=[
            pl.BlockSpec((rows, D_HID), lambda i: (i, 0)),
            pl.BlockSpec((rows, D_HID), lambda i: (i, 0)),
        ],
        out_specs=pl.BlockSpec((rows, D_HID), lambda i: (i, 0)),
        out_shape=jax.ShapeDtypeStruct((N_PAD, D_HID), jnp.float32),
    )(p0, p1)


def _mm2_body(h_ref, w_ref, o_ref):
    o_ref[...] = jnp.dot(h_ref[...], w_ref[...],
                         preferred_element_type=jnp.float32)


def _mm2(h16, W2):
    return pl.pallas_call(
        _mm2_body,
        in_specs=[
            pl.BlockSpec((N_IDX, D_HID), lambda: (0, 0)),
            pl.BlockSpec((D_HID, D_OUT), lambda: (0, 0)),
        ],
        out_specs=pl.BlockSpec((N_IDX, D_OUT), lambda: (0, 0)),
        out_shape=jax.ShapeDtypeStruct((N_IDX, D_OUT), jnp.float32),
    )(h16, W2)


def kernel(x, edge_index, edge_weight, index, W1, W2):
    shp = (NC, NS, N_CHUNKS, CH)
    row = edge_index[:, 0].reshape(shp)
    col = edge_index[:, 1].reshape(shp)
    edge_weight = edge_weight.reshape(shp)
    hw1 = _mm1(x, W1)
    p0, p1 = _agg(hw1, row, col, edge_weight)
    h = _comb(p0, p1)
    q0, q1 = _agg(h, row, col, edge_weight)
    out16 = _fin(q0, q1, index)
    return _mm2(out16, W2)


# speedup vs baseline: 25.2374x; 1.1489x over previous
import functools

import jax
import jax.numpy as jnp
from jax import lax
from jax.experimental import pallas as pl
from jax.experimental.pallas import tpu as pltpu
from jax.experimental.pallas import tpu_sc as plsc

N_NODES = 10000
N_EDGES = 320000
D_IN = 128
D_HID = 16
D_OUT = 64
N_IDX = 1024
N_PAD = 10240

NC = 2
NS = 16
CH = 80

E_PER_CORE = N_EDGES // NC
E_PER_TILE = E_PER_CORE // NS
N_CHUNKS = E_PER_TILE // CH
ROWS_PER_TILE = N_PAD // NS
IDX_PER_TILE = N_IDX // (NC * NS)

_mesh = functools.partial(
    plsc.VectorSubcoreMesh,
    core_axis_name="c",
    subcore_axis_name="s",
    num_cores=NC,
    num_subcores=NS,
)


NBUF = 5
N_GROUPS = N_CHUNKS // NBUF


def _zero_acc(zbuf, acc, sl):
    def zero_row(i, carry):
        zbuf[i, :] = jnp.zeros((16,), jnp.float32)
        return carry
    lax.fori_loop(0, ROWS_PER_TILE, zero_row, 0)
    pltpu.sync_copy(zbuf, acc.at[sl])


def _edge_loop(hsrc, colv, roww, wv, bufa, bufb, acc, gsa, gsb, ssa, ssb):
    def fire_g(j0, bufs, sems):
        return [
            pltpu.async_copy(hsrc.at[colv.at[j0 + r]], bufs.at[r], sems[r])
            for r in range(NBUF)
        ]

    def fire_s(j0, bufs, sems):
        return [
            pltpu.async_copy(bufs.at[r], acc.at[roww.at[j0 + r]], sems[r],
                             add=True)
            for r in range(NBUF)
        ]

    def scale_group(j0, bufs):
        for r in range(NBUF):
            def scale(m, inner, r=r):
                w16 = wv[j0 + r, pl.ds(m * 16, 16)]
                e0 = m * 16
                for k in range(16):
                    bufs[r, e0 + k, :] = bufs[r, e0 + k, :] * w16[k]
                return inner
            lax.fori_loop(0, CH // 16, scale, 0)

    def pair(p, carry):
        j0 = (2 * p) * NBUF
        j1 = j0 + NBUF
        da = fire_g(j0, bufa, gsa)
        db = fire_g(j1, bufb, gsb)
        for d in da:
            d.wait()
        scale_group(j0, bufa)
        sa = fire_s(j0, bufa, ssa)
        for d in db:
            d.wait()
        scale_group(j1, bufb)
        sb = fire_s(j1, bufb, ssb)
        for d in sa + sb:
            d.wait()
        return carry

    lax.fori_loop(0, N_GROUPS // 2, pair, 0)
    jt = (N_GROUPS - 1) * NBUF
    da = fire_g(jt, bufa, gsa)
    for d in da:
        d.wait()
    scale_group(jt, bufa)
    for d in fire_s(jt, bufa, ssa):
        d.wait()


def _drain(acc, sl, c, out0_hbm, out1_hbm):
    @pl.when(c == 0)
    def _():
        pltpu.sync_copy(acc.at[sl], out0_hbm.at[sl])

    @pl.when(c == 1)
    def _():
        pltpu.sync_copy(acc.at[sl], out1_hbm.at[sl])


_AGG_SCRATCH = [
    pltpu.VMEM((N_CHUNKS, CH), jnp.int32),
    pltpu.VMEM((N_CHUNKS, CH), jnp.int32),
    pltpu.VMEM((N_CHUNKS, CH), jnp.float32),
    pltpu.VMEM((NBUF, CH, D_HID), jnp.float32),
    pltpu.VMEM((NBUF, CH, D_HID), jnp.float32),
    pltpu.VMEM((ROWS_PER_TILE, D_HID), jnp.float32),
    pltpu.VMEM_SHARED((N_PAD, D_HID), jnp.float32),
]


def _agg1_body(src_hbm, row_hbm, col_hbm, w_hbm, out0_hbm, out1_hbm,
               colv, roww, wv, bufa, bufb, zbuf, acc, *sems):
    gsa, gsb, ssa, ssb = (sems[i * NBUF:(i + 1) * NBUF] for i in range(4))
    c = lax.axis_index("c")
    s = lax.axis_index("s")
    sl = pl.ds(s * ROWS_PER_TILE, ROWS_PER_TILE)

    _zero_acc(zbuf, acc, sl)
    pltpu.sync_copy(col_hbm.at[c, s], colv)
    pltpu.sync_copy(row_hbm.at[c, s], roww)
    pltpu.sync_copy(w_hbm.at[c, s], wv)
    plsc.subcore_barrier()
    _edge_loop(src_hbm, colv, roww, wv, bufa, bufb, acc, gsa, gsb, ssa, ssb)
    plsc.subcore_barrier()
    _drain(acc, sl, c, out0_hbm, out1_hbm)


_agg1 = pl.kernel(
    _agg1_body,
    out_type=(
        jax.ShapeDtypeStruct((N_PAD, D_HID), jnp.float32),
        jax.ShapeDtypeStruct((N_PAD, D_HID), jnp.float32),
    ),
    mesh=_mesh(),
    scratch_types=_AGG_SCRATCH + [pltpu.SemaphoreType.DMA] * (4 * NBUF),
    compiler_params=pltpu.CompilerParams(use_tc_tiling_on_sc=False),
    name="edge_agg1_sc",
)


def _agg2_body(p0_hbm, p1_hbm, row_hbm, col_hbm, w_hbm,
               out0_hbm, out1_hbm, h_hbm,
               colv, roww, wv, bufa, bufb, zbuf, hbuf, acc, *sems):
    gsa, gsb, ssa, ssb = (sems[i * NBUF:(i + 1) * NBUF] for i in range(4))
    c = lax.axis_index("c")
    s = lax.axis_index("s")
    sl = pl.ds(s * ROWS_PER_TILE, ROWS_PER_TILE)

    _zero_acc(zbuf, acc, sl)
    pltpu.sync_copy(col_hbm.at[c, s], colv)
    pltpu.sync_copy(row_hbm.at[c, s], roww)
    pltpu.sync_copy(w_hbm.at[c, s], wv)

    pltpu.sync_copy(p0_hbm.at[sl], zbuf)
    pltpu.sync_copy(p1_hbm.at[sl], hbuf)

    def comb_row(i, carry):
        zbuf[i, :] = jnp.maximum(zbuf[i, :] + hbuf[i, :], 0.0)
        return carry
    lax.fori_loop(0, ROWS_PER_TILE, comb_row, 0)
    pltpu.sync_copy(zbuf, h_hbm.at[c, sl])
    plsc.subcore_barrier()

    hsrc = h_hbm.at[c]
    _edge_loop(hsrc, colv, roww, wv, bufa, bufb, acc, gsa, gsb, ssa, ssb)
    plsc.subcore_barrier()
    _drain(acc, sl, c, out0_hbm, out1_hbm)


_agg2 = pl.kernel(
    _agg2_body,
    out_type=(
        jax.ShapeDtypeStruct((N_PAD, D_HID), jnp.float32),
        jax.ShapeDtypeStruct((N_PAD, D_HID), jnp.float32),
        jax.ShapeDtypeStruct((NC, N_PAD, D_HID), jnp.float32),
    ),
    mesh=_mesh(),
    scratch_types=_AGG_SCRATCH[:5]
    + [pltpu.VMEM((ROWS_PER_TILE, D_HID), jnp.float32)] * 2
    + [_AGG_SCRATCH[6]]
    + [pltpu.SemaphoreType.DMA] * (4 * NBUF),
    compiler_params=pltpu.CompilerParams(use_tc_tiling_on_sc=False),
    name="edge_agg2_sc",
)


def _fin_body(q0_hbm, q1_hbm, idx_hbm, out_hbm, idxv, a_v, b_v, sem):
    wid = lax.axis_index("s") * NC + lax.axis_index("c")
    base = wid * IDX_PER_TILE
    pltpu.sync_copy(idx_hbm.at[pl.ds(base, IDX_PER_TILE)], idxv)
    pltpu.async_copy(q0_hbm.at[idxv], a_v, sem).wait()
    pltpu.async_copy(q1_hbm.at[idxv], b_v, sem).wait()

    def add_row(i, carry):
        a_v[i, :] = a_v[i, :] + b_v[i, :]
        return carry
    lax.fori_loop(0, IDX_PER_TILE, add_row, 0)
    pltpu.sync_copy(a_v, out_hbm.at[pl.ds(base, IDX_PER_TILE)])


_fin = pl.kernel(
    _fin_body,
    out_type=jax.ShapeDtypeStruct((N_IDX, D_HID), jnp.float32),
    mesh=_mesh(),
    scratch_types=[
        pltpu.VMEM((IDX_PER_TILE,), jnp.int32),
        pltpu.VMEM((IDX_PER_TILE, D_HID), jnp.float32),
        pltpu.VMEM((IDX_PER_TILE, D_HID), jnp.float32),
        pltpu.SemaphoreType.DMA,
    ],
    compiler_params=pltpu.CompilerParams(use_tc_tiling_on_sc=False),
    name="final_gather_sc",
)


def _mm1_body(x_ref, w_ref, o_ref):
    o_ref[...] = jnp.dot(x_ref[...], w_ref[...],
                         preferred_element_type=jnp.float32)


def _mm1(x, W1):
    grid = 5
    rows = N_NODES // grid
    return pl.pallas_call(
        _mm1_body,
        grid=(grid,),
        in_specs=[
            pl.BlockSpec((rows, D_IN), lambda i: (i, 0)),
            pl.BlockSpec((D_IN, D_HID), lambda i: (0, 0)),
        ],
        out_specs=pl.BlockSpec((rows, D_HID), lambda i: (i, 0)),
        out_shape=jax.ShapeDtypeStruct((N_NODES, D_HID), jnp.float32),
    )(x, W1)


def _mm2_body(h_ref, w_ref, o_ref):
    o_ref[...] = jnp.dot(h_ref[...], w_ref[...],
                         preferred_element_type=jnp.float32)


def _mm2(h16, W2):
    return pl.pallas_call(
        _mm2_body,
        in_specs=[
            pl.BlockSpec((N_IDX, D_HID), lambda: (0, 0)),
            pl.BlockSpec((D_HID, D_OUT), lambda: (0, 0)),
        ],
        out_specs=pl.BlockSpec((N_IDX, D_OUT), lambda: (0, 0)),
        out_shape=jax.ShapeDtypeStruct((N_IDX, D_OUT), jnp.float32),
    )(h16, W2)


def kernel(x, edge_index, edge_weight, index, W1, W2):
    shp = (NC, NS, N_CHUNKS, CH)
    row = edge_index[:, 0].reshape(shp)
    col = edge_index[:, 1].reshape(shp)
    edge_weight = edge_weight.reshape(shp)
    hw1 = _mm1(x, W1)
    p0, p1 = _agg1(hw1, row, col, edge_weight)
    q0, q1, _ = _agg2(p0, p1, row, col, edge_weight)
    out16 = _fin(q0, q1, index)
    return _mm2(out16, W2)
